# baseline (speedup 1.0000x reference)
import functools

import jax
import jax.numpy as jnp
from jax import lax
from jax.experimental import pallas as pl
from jax.experimental.pallas import tpu as pltpu
from jax.experimental.pallas import tpu_sc as plsc

_B = 16384
_D = 20
_EPS = 1e-3
_NW = 32
_BPW = _B // _NW
_K = 16
_NGRP = _BPW // _K
_NSLAB = 125000


def _sc_gather_one(slab, sub, tab3):
    mesh = plsc.VectorSubcoreMesh(core_axis_name="c", subcore_axis_name="s")

    @functools.partial(
        pl.kernel,
        mesh=mesh,
        out_type=jax.ShapeDtypeStruct((_B, 1, _D), jnp.float32),
        scratch_types=[
            pltpu.MemorySpace.VMEM_SHARED((_NW, _BPW), jnp.int32),
            pltpu.MemorySpace.VMEM_SHARED((_NW, _BPW), jnp.int32),
            pltpu.SMEM((_BPW,), jnp.int32),
            pltpu.SMEM((_BPW,), jnp.int32),
            pltpu.VMEM((_BPW, 1, _D), jnp.float32),
            pltpu.SemaphoreType.DMA,
        ],
    )
    def gather_kernel(slab_hbm, sub_hbm, tab_hbm, out_hbm,
                      slab_sh, sub_sh, slab_s, sub_s, row_v, sem):
        wid = lax.axis_index("s") * 2 + lax.axis_index("c")
        base = wid * _BPW
        pltpu.sync_copy(slab_hbm.at[pl.ds(base, _BPW)], slab_sh.at[wid])
        pltpu.sync_copy(sub_hbm.at[pl.ds(base, _BPW)], sub_sh.at[wid])
        pltpu.sync_copy(slab_sh.at[wid], slab_s)
        pltpu.sync_copy(sub_sh.at[wid], sub_s)

        def body(g, carry):
            cps = []
            for b in range(_K):
                i = g * _K + b
                cps.append(pltpu.async_copy(
                    tab_hbm.at[pl.ds(slab_s[i], 1), pl.ds(sub_s[i], 1)],
                    row_v.at[pl.ds(i, 1)], sem))
            for cp in cps:
                cp.wait()
            return carry

        lax.fori_loop(0, _NGRP, body, 0)
        pltpu.sync_copy(row_v, out_hbm.at[pl.ds(base, _BPW)])

    return gather_kernel(slab, sub, tab3)


def _mlp_body(pv_ref, cv_ref,
              pw_ref, pb_ref, pg_ref, pbb_ref, pm_ref, pvv_ref,
              cw_ref, cb_ref, cg_ref, cbb_ref, cm_ref, cvv_ref,
              w1p_ref, w1c_ref, b1_ref, g1_ref, bb1_ref, m1_ref, v1_ref,
              wo_ref, bo_ref, o_ref):
    blk = pv_ref.shape[0]
    pv = pv_ref[...].reshape(blk, _D)
    cv = cv_ref[...].reshape(blk, _D)

    psc = pg_ref[...] / jnp.sqrt(pvv_ref[...] + _EPS)
    psh = pbb_ref[...] - pm_ref[...] * psc
    csc = cg_ref[...] / jnp.sqrt(cvv_ref[...] + _EPS)
    csh = cbb_ref[...] - cm_ref[...] * csc
    s1 = g1_ref[...] / jnp.sqrt(v1_ref[...] + _EPS)
    t1 = bb1_ref[...] - m1_ref[...] * s1

    ph = jnp.maximum(
        jnp.dot(pv, pw_ref[...], preferred_element_type=jnp.float32)
        + pb_ref[...], 0.0) * psc + psh
    ch = jnp.maximum(
        jnp.dot(cv, cw_ref[...], preferred_element_type=jnp.float32)
        + cb_ref[...], 0.0) * csc + csh
    z = (jnp.dot(ph, w1p_ref[...], preferred_element_type=jnp.float32)
         + jnp.dot(ch, w1c_ref[...], preferred_element_type=jnp.float32)
         + b1_ref[...])
    h = jnp.maximum(z, 0.0) * s1 + t1
    logit = jnp.dot(h, wo_ref[...], preferred_element_type=jnp.float32) + bo_ref[...]
    o_ref[...] = 1.0 / (1.0 + jnp.exp(-logit))


def _tc_mlp(pv, cv,
            pw, pb, pg, pbb, pm, pvv, cw, cb, cg, cbb, cm, cvv,
            w1p, w1c, b1, g1, bb1, m1, v1, wo, bo):
    blk = 2048
    grid = (_B // blk,)
    row_spec = pl.BlockSpec((blk, 1, _D), lambda i: (i, 0, 0))

    def full(a):
        return pl.BlockSpec(a.shape, lambda i: tuple(0 for _ in a.shape))

    weights = (pw, pb, pg, pbb, pm, pvv, cw, cb, cg, cbb, cm, cvv,
               w1p, w1c, b1, g1, bb1, m1, v1, wo, bo)
    return pl.pallas_call(
        _mlp_body,
        grid=grid,
        in_specs=[row_spec, row_spec] + [full(w) for w in weights],
        out_specs=pl.BlockSpec((blk, 1), lambda i: (i, 0)),
        out_shape=jax.ShapeDtypeStruct((_B, 1), jnp.float32),
    )(pv, cv, *weights)


def kernel(X, prod_emb, cust_emb, prod_fc1_w, prod_fc1_b, prod_bn_g,
           prod_bn_b, prod_bn_m, prod_bn_v, cust_fc1_w, cust_fc1_b,
           cust_bn_g, cust_bn_b, cust_bn_m, cust_bn_v, fc1_w, fc1_b,
           fc1_bn_g, fc1_bn_b, fc1_bn_m, fc1_bn_v, out_w, out_b):
    pidx = X[:, 0].astype(jnp.int32)
    cidx = X[:, 1].astype(jnp.int32)
    pt3 = prod_emb.reshape(_NSLAB, 8, _D)
    ct3 = cust_emb.reshape(_NSLAB, 8, _D)
    pv = _sc_gather_one(pidx >> 3, pidx & 7, pt3)
    cv = _sc_gather_one(cidx >> 3, cidx & 7, ct3)

    r2 = lambda a: a.reshape(1, -1)
    return _tc_mlp(
        pv, cv,
        prod_fc1_w, r2(prod_fc1_b), r2(prod_bn_g), r2(prod_bn_b),
        r2(prod_bn_m), r2(prod_bn_v),
        cust_fc1_w, r2(cust_fc1_b), r2(cust_bn_g), r2(cust_bn_b),
        r2(cust_bn_m), r2(cust_bn_v),
        fc1_w[:10, :], fc1_w[10:, :], r2(fc1_b), r2(fc1_bn_g),
        r2(fc1_bn_b), r2(fc1_bn_m), r2(fc1_bn_v),
        out_w, r2(out_b),
    )


# speedup vs baseline: 1.4856x; 1.4856x over previous
import functools

import jax
import jax.numpy as jnp
from jax import lax
from jax.experimental import pallas as pl
from jax.experimental.pallas import tpu as pltpu
from jax.experimental.pallas import tpu_sc as plsc

_B = 16384
_D = 20
_EPS = 1e-3
_NW = 32
_BPW = _B // _NW
_K = 16
_NGRP = _BPW // _K
_NSLAB = 125000


def _sc_gather_one(slab, sub, tab3):
    mesh = plsc.VectorSubcoreMesh(core_axis_name="c", subcore_axis_name="s")

    @functools.partial(
        pl.kernel,
        mesh=mesh,
        out_type=jax.ShapeDtypeStruct((_B, 1, _D), jnp.float32),
        scratch_types=[
            pltpu.MemorySpace.VMEM_SHARED((_NW, _BPW), jnp.int32),
            pltpu.MemorySpace.VMEM_SHARED((_NW, _BPW), jnp.int32),
            pltpu.SMEM((_BPW,), jnp.int32),
            pltpu.SMEM((_BPW,), jnp.int32),
            pltpu.VMEM((_BPW, 1, _D), jnp.float32),
            pltpu.SemaphoreType.DMA,
        ],
    )
    def gather_kernel(slab_hbm, sub_hbm, tab_hbm, out_hbm,
                      slab_sh, sub_sh, slab_s, sub_s, row_v, sem):
        wid = lax.axis_index("s") * 2 + lax.axis_index("c")
        base = wid * _BPW
        pltpu.sync_copy(slab_hbm.at[pl.ds(base, _BPW)], slab_sh.at[wid])
        pltpu.sync_copy(sub_hbm.at[pl.ds(base, _BPW)], sub_sh.at[wid])
        pltpu.sync_copy(slab_sh.at[wid], slab_s)
        pltpu.sync_copy(sub_sh.at[wid], sub_s)

        def body(g, carry):
            cps = []
            for b in range(_K):
                i = g * _K + b
                cps.append(pltpu.async_copy(
                    tab_hbm.at[pl.ds(slab_s[i], 1), pl.ds(sub_s[i], 1)],
                    row_v.at[pl.ds(i, 1)], sem))
            for cp in cps:
                cp.wait()
            return carry

        lax.fori_loop(0, _NGRP, body, 0)
        pltpu.sync_copy(row_v, out_hbm.at[pl.ds(base, _BPW)])

    return gather_kernel(slab, sub, tab3)


def _sc_gather_2d(idx, tab):
    mesh = plsc.VectorSubcoreMesh(core_axis_name="c", subcore_axis_name="s")

    @functools.partial(
        pl.kernel,
        mesh=mesh,
        out_type=jax.ShapeDtypeStruct((_B, _D), jnp.float32),
        scratch_types=[
            pltpu.MemorySpace.VMEM_SHARED((_NW, _BPW), jnp.int32),
            pltpu.SMEM((_BPW,), jnp.int32),
            pltpu.VMEM((_BPW, _D), jnp.float32),
            pltpu.SemaphoreType.DMA,
        ],
    )
    def gather_kernel(idx_hbm, tab_hbm, out_hbm, idx_sh, idx_s, row_v, sem):
        wid = lax.axis_index("s") * 2 + lax.axis_index("c")
        base = wid * _BPW
        pltpu.sync_copy(idx_hbm.at[pl.ds(base, _BPW)], idx_sh.at[wid])
        pltpu.sync_copy(idx_sh.at[wid], idx_s)

        def body(g, carry):
            cps = []
            for b in range(_K):
                i = g * _K + b
                cps.append(pltpu.async_copy(
                    tab_hbm.at[pl.ds(idx_s[i], 1)],
                    row_v.at[pl.ds(i, 1)], sem))
            for cp in cps:
                cp.wait()
            return carry

        lax.fori_loop(0, _NGRP, body, 0)
        pltpu.sync_copy(row_v, out_hbm.at[pl.ds(base, _BPW)])

    return gather_kernel(idx, tab)


def _mlp_body(pv_ref, cv_ref,
              pw_ref, pb_ref, pg_ref, pbb_ref, pm_ref, pvv_ref,
              cw_ref, cb_ref, cg_ref, cbb_ref, cm_ref, cvv_ref,
              w1p_ref, w1c_ref, b1_ref, g1_ref, bb1_ref, m1_ref, v1_ref,
              wo_ref, bo_ref, o_ref):
    blk = pv_ref.shape[0]
    pv = pv_ref[...].reshape(blk, _D)
    cv = cv_ref[...].reshape(blk, _D)

    psc = pg_ref[...] / jnp.sqrt(pvv_ref[...] + _EPS)
    psh = pbb_ref[...] - pm_ref[...] * psc
    csc = cg_ref[...] / jnp.sqrt(cvv_ref[...] + _EPS)
    csh = cbb_ref[...] - cm_ref[...] * csc
    s1 = g1_ref[...] / jnp.sqrt(v1_ref[...] + _EPS)
    t1 = bb1_ref[...] - m1_ref[...] * s1

    ph = jnp.maximum(
        jnp.dot(pv, pw_ref[...], preferred_element_type=jnp.float32)
        + pb_ref[...], 0.0) * psc + psh
    ch = jnp.maximum(
        jnp.dot(cv, cw_ref[...], preferred_element_type=jnp.float32)
        + cb_ref[...], 0.0) * csc + csh
    z = (jnp.dot(ph, w1p_ref[...], preferred_element_type=jnp.float32)
         + jnp.dot(ch, w1c_ref[...], preferred_element_type=jnp.float32)
         + b1_ref[...])
    h = jnp.maximum(z, 0.0) * s1 + t1
    logit = jnp.dot(h, wo_ref[...], preferred_element_type=jnp.float32) + bo_ref[...]
    o_ref[...] = 1.0 / (1.0 + jnp.exp(-logit))


def _tc_mlp(pv, cv,
            pw, pb, pg, pbb, pm, pvv, cw, cb, cg, cbb, cm, cvv,
            w1p, w1c, b1, g1, bb1, m1, v1, wo, bo):
    blk = 2048
    grid = (_B // blk,)
    row_spec3 = pl.BlockSpec((blk, 1, _D), lambda i: (i, 0, 0))
    row_spec2 = pl.BlockSpec((blk, _D), lambda i: (i, 0))

    def full(a):
        return pl.BlockSpec(a.shape, lambda i: tuple(0 for _ in a.shape))

    weights = (pw, pb, pg, pbb, pm, pvv, cw, cb, cg, cbb, cm, cvv,
               w1p, w1c, b1, g1, bb1, m1, v1, wo, bo)
    return pl.pallas_call(
        _mlp_body,
        grid=grid,
        in_specs=[row_spec3, row_spec2] + [full(w) for w in weights],
        out_specs=pl.BlockSpec((blk, 1), lambda i: (i, 0)),
        out_shape=jax.ShapeDtypeStruct((_B, 1), jnp.float32),
    )(pv, cv, *weights)


def kernel(X, prod_emb, cust_emb, prod_fc1_w, prod_fc1_b, prod_bn_g,
           prod_bn_b, prod_bn_m, prod_bn_v, cust_fc1_w, cust_fc1_b,
           cust_bn_g, cust_bn_b, cust_bn_m, cust_bn_v, fc1_w, fc1_b,
           fc1_bn_g, fc1_bn_b, fc1_bn_m, fc1_bn_v, out_w, out_b):
    pidx = X[:, 0].astype(jnp.int32)
    cidx = X[:, 1].astype(jnp.int32)
    pt3 = prod_emb.reshape(_NSLAB, 8, _D)
    pv = _sc_gather_one(pidx >> 3, pidx & 7, pt3)
    cv = _sc_gather_2d(cidx, cust_emb)

    r2 = lambda a: a.reshape(1, -1)
    return _tc_mlp(
        pv, cv,
        prod_fc1_w, r2(prod_fc1_b), r2(prod_bn_g), r2(prod_bn_b),
        r2(prod_bn_m), r2(prod_bn_v),
        cust_fc1_w, r2(cust_fc1_b), r2(cust_bn_g), r2(cust_bn_b),
        r2(cust_bn_m), r2(cust_bn_v),
        fc1_w[:10, :], fc1_w[10:, :], r2(fc1_b), r2(fc1_bn_g),
        r2(fc1_bn_b), r2(fc1_bn_m), r2(fc1_bn_v),
        out_w, r2(out_b),
    )


# speedup vs baseline: 1.5453x; 1.0402x over previous
import functools

import jax
import jax.numpy as jnp
from jax import lax
from jax.experimental import pallas as pl
from jax.experimental.pallas import tpu as pltpu
from jax.experimental.pallas import tpu_sc as plsc

_B = 16384
_D = 20
_EPS = 1e-3
_NW = 32
_BPW = _B // _NW
_K = 32
_NGRP = _BPW // _K
_NSLAB = 125000


def _sc_gather_one(slab, sub, tab3):
    mesh = plsc.VectorSubcoreMesh(core_axis_name="c", subcore_axis_name="s")

    @functools.partial(
        pl.kernel,
        mesh=mesh,
        out_type=jax.ShapeDtypeStruct((_B, 1, _D), jnp.float32),
        scratch_types=[
            pltpu.MemorySpace.VMEM_SHARED((_NW, _BPW), jnp.int32),
            pltpu.MemorySpace.VMEM_SHARED((_NW, _BPW), jnp.int32),
            pltpu.SMEM((_BPW,), jnp.int32),
            pltpu.SMEM((_BPW,), jnp.int32),
            pltpu.VMEM((_BPW, 1, _D), jnp.float32),
            pltpu.SemaphoreType.DMA,
        ],
    )
    def gather_kernel(slab_hbm, sub_hbm, tab_hbm, out_hbm,
                      slab_sh, sub_sh, slab_s, sub_s, row_v, sem):
        wid = lax.axis_index("s") * 2 + lax.axis_index("c")
        base = wid * _BPW
        pltpu.sync_copy(slab_hbm.at[pl.ds(base, _BPW)], slab_sh.at[wid])
        pltpu.sync_copy(sub_hbm.at[pl.ds(base, _BPW)], sub_sh.at[wid])
        pltpu.sync_copy(slab_sh.at[wid], slab_s)
        pltpu.sync_copy(sub_sh.at[wid], sub_s)

        def body(g, carry):
            cps = []
            for b in range(_K):
                i = g * _K + b
                cps.append(pltpu.async_copy(
                    tab_hbm.at[pl.ds(slab_s[i], 1), pl.ds(sub_s[i], 1)],
                    row_v.at[pl.ds(i, 1)], sem))
            for cp in cps:
                cp.wait()
            return carry

        lax.fori_loop(0, _NGRP, body, 0)
        pltpu.sync_copy(row_v, out_hbm.at[pl.ds(base, _BPW)])

    return gather_kernel(slab, sub, tab3)


def _sc_gather_2d(idx, tab):
    mesh = plsc.VectorSubcoreMesh(core_axis_name="c", subcore_axis_name="s")

    @functools.partial(
        pl.kernel,
        mesh=mesh,
        out_type=jax.ShapeDtypeStruct((_B, _D), jnp.float32),
        scratch_types=[
            pltpu.MemorySpace.VMEM_SHARED((_NW, _BPW), jnp.int32),
            pltpu.SMEM((_BPW,), jnp.int32),
            pltpu.VMEM((_BPW, _D), jnp.float32),
            pltpu.SemaphoreType.DMA,
        ],
    )
    def gather_kernel(idx_hbm, tab_hbm, out_hbm, idx_sh, idx_s, row_v, sem):
        wid = lax.axis_index("s") * 2 + lax.axis_index("c")
        base = wid * _BPW
        pltpu.sync_copy(idx_hbm.at[pl.ds(base, _BPW)], idx_sh.at[wid])
        pltpu.sync_copy(idx_sh.at[wid], idx_s)

        def body(g, carry):
            cps = []
            for b in range(_K):
                i = g * _K + b
                cps.append(pltpu.async_copy(
                    tab_hbm.at[pl.ds(idx_s[i], 1)],
                    row_v.at[pl.ds(i, 1)], sem))
            for cp in cps:
                cp.wait()
            return carry

        lax.fori_loop(0, _NGRP, body, 0)
        pltpu.sync_copy(row_v, out_hbm.at[pl.ds(base, _BPW)])

    return gather_kernel(idx, tab)


def _mlp_body(pv_ref, cv_ref,
              pw_ref, pb_ref, pg_ref, pbb_ref, pm_ref, pvv_ref,
              cw_ref, cb_ref, cg_ref, cbb_ref, cm_ref, cvv_ref,
              w1p_ref, w1c_ref, b1_ref, g1_ref, bb1_ref, m1_ref, v1_ref,
              wo_ref, bo_ref, o_ref):
    blk = pv_ref.shape[0]
    pv = pv_ref[...].reshape(blk, _D)
    cv = cv_ref[...].reshape(blk, _D)

    psc = pg_ref[...] / jnp.sqrt(pvv_ref[...] + _EPS)
    psh = pbb_ref[...] - pm_ref[...] * psc
    csc = cg_ref[...] / jnp.sqrt(cvv_ref[...] + _EPS)
    csh = cbb_ref[...] - cm_ref[...] * csc
    s1 = g1_ref[...] / jnp.sqrt(v1_ref[...] + _EPS)
    t1 = bb1_ref[...] - m1_ref[...] * s1

    ph = jnp.maximum(
        jnp.dot(pv, pw_ref[...], preferred_element_type=jnp.float32)
        + pb_ref[...], 0.0) * psc + psh
    ch = jnp.maximum(
        jnp.dot(cv, cw_ref[...], preferred_element_type=jnp.float32)
        + cb_ref[...], 0.0) * csc + csh
    z = (jnp.dot(ph, w1p_ref[...], preferred_element_type=jnp.float32)
         + jnp.dot(ch, w1c_ref[...], preferred_element_type=jnp.float32)
         + b1_ref[...])
    h = jnp.maximum(z, 0.0) * s1 + t1
    logit = jnp.dot(h, wo_ref[...], preferred_element_type=jnp.float32) + bo_ref[...]
    o_ref[...] = 1.0 / (1.0 + jnp.exp(-logit))


def _tc_mlp(pv, cv,
            pw, pb, pg, pbb, pm, pvv, cw, cb, cg, cbb, cm, cvv,
            w1p, w1c, b1, g1, bb1, m1, v1, wo, bo):
    blk = 2048
    grid = (_B // blk,)
    row_spec3 = pl.BlockSpec((blk, 1, _D), lambda i: (i, 0, 0))
    row_spec2 = pl.BlockSpec((blk, _D), lambda i: (i, 0))

    def full(a):
        return pl.BlockSpec(a.shape, lambda i: tuple(0 for _ in a.shape))

    weights = (pw, pb, pg, pbb, pm, pvv, cw, cb, cg, cbb, cm, cvv,
               w1p, w1c, b1, g1, bb1, m1, v1, wo, bo)
    return pl.pallas_call(
        _mlp_body,
        grid=grid,
        in_specs=[row_spec3, row_spec2] + [full(w) for w in weights],
        out_specs=pl.BlockSpec((blk, 1), lambda i: (i, 0)),
        out_shape=jax.ShapeDtypeStruct((_B, 1), jnp.float32),
    )(pv, cv, *weights)


def kernel(X, prod_emb, cust_emb, prod_fc1_w, prod_fc1_b, prod_bn_g,
           prod_bn_b, prod_bn_m, prod_bn_v, cust_fc1_w, cust_fc1_b,
           cust_bn_g, cust_bn_b, cust_bn_m, cust_bn_v, fc1_w, fc1_b,
           fc1_bn_g, fc1_bn_b, fc1_bn_m, fc1_bn_v, out_w, out_b):
    pidx = X[:, 0].astype(jnp.int32)
    cidx = X[:, 1].astype(jnp.int32)
    pt3 = prod_emb.reshape(_NSLAB, 8, _D)
    pv = _sc_gather_one(pidx >> 3, pidx & 7, pt3)
    cv = _sc_gather_2d(cidx, cust_emb)

    r2 = lambda a: a.reshape(1, -1)
    return _tc_mlp(
        pv, cv,
        prod_fc1_w, r2(prod_fc1_b), r2(prod_bn_g), r2(prod_bn_b),
        r2(prod_bn_m), r2(prod_bn_v),
        cust_fc1_w, r2(cust_fc1_b), r2(cust_bn_g), r2(cust_bn_b),
        r2(cust_bn_m), r2(cust_bn_v),
        fc1_w[:10, :], fc1_w[10:, :], r2(fc1_b), r2(fc1_bn_g),
        r2(fc1_bn_b), r2(fc1_bn_m), r2(fc1_bn_v),
        out_w, r2(out_b),
    )
